# baseline (speedup 1.0000x reference)
import jax
import jax.numpy as jnp
from jax.experimental import pallas as pl
from jax.experimental.pallas import tpu as pltpu

_DIM = 32
_CODES = 8192
_COMMIT_W = 0.25


def _vq_tile_kernel(x_ref, emb_ref, q_ref, idx_ref, loss_ref):
    x = x_ref[...]
    emb = emb_ref[...]
    x2 = jnp.sum(x * x, axis=1, keepdims=True)
    e2 = jnp.sum(emb * emb, axis=1)
    mm = jax.lax.dot_general(
        x.astype(jnp.bfloat16), emb.astype(jnp.bfloat16),
        (((1,), (1,)), ((), ())),
        preferred_element_type=jnp.float32)
    d = (x2 + e2[None, :]) - 2.0 * mm
    idx = jnp.argmin(d, axis=1).astype(jnp.int32)
    code_iota = jax.lax.broadcasted_iota(jnp.int32, d.shape, 1)
    onehot = (code_iota == idx[:, None]).astype(jnp.bfloat16)
    e_hi = emb.astype(jnp.bfloat16)
    e_lo = (emb - e_hi.astype(jnp.float32)).astype(jnp.bfloat16)
    dn = (((1,), (0,)), ((), ()))
    q = (jax.lax.dot_general(onehot, e_hi, dn,
                             preferred_element_type=jnp.float32)
         + jax.lax.dot_general(onehot, e_lo, dn,
                               preferred_element_type=jnp.float32))
    idx_ref[0, 0, :] = idx
    q_ref[...] = x + (q - x)
    diff = x - q
    part = jnp.sum(diff * diff)

    @pl.when(pl.program_id(0) == 0)
    def _():
        loss_ref[0, 0] = 0.0

    loss_ref[0, 0] += part


def kernel(x, emb):
    b, s, d = x.shape
    n = b * s
    tile = 512
    grid = n // tile
    x_flat = x.reshape(n, d)
    q_flat, idx2d, loss_sum = pl.pallas_call(
        _vq_tile_kernel,
        grid=(grid,),
        in_specs=[
            pl.BlockSpec((tile, d), lambda i: (i, 0)),
            pl.BlockSpec((_CODES, d), lambda i: (0, 0)),
        ],
        out_specs=[
            pl.BlockSpec((tile, d), lambda i: (i, 0)),
            pl.BlockSpec((1, 1, tile), lambda i: (i, 0, 0)),
            pl.BlockSpec(memory_space=pltpu.SMEM),
        ],
        out_shape=[
            jax.ShapeDtypeStruct((n, d), jnp.float32),
            jax.ShapeDtypeStruct((grid, 1, tile), jnp.int32),
            jax.ShapeDtypeStruct((1, 1), jnp.float32),
        ],
    )(x_flat, emb)
    quantized = q_flat.reshape(b, s, d)
    indices = idx2d.reshape(b, s)
    m = loss_sum[0, 0] / jnp.float32(n * d)
    loss = m + _COMMIT_W * m
    return (quantized, indices, loss)


# speedup vs baseline: 1.3058x; 1.3058x over previous
import jax
import jax.numpy as jnp
from jax.experimental import pallas as pl
from jax.experimental.pallas import tpu as pltpu

_DIM = 32
_CODES = 8192
_COMMIT_W = 0.25


def _vq_tile_kernel(x_ref, emb_ref, q_ref, idx_ref, loss_ref):
    x = x_ref[...]
    emb = emb_ref[...]
    x2 = jnp.sum(x * x, axis=1, keepdims=True)
    e2 = jnp.sum(emb * emb, axis=1)
    mm = jax.lax.dot_general(
        x.astype(jnp.bfloat16), emb.astype(jnp.bfloat16),
        (((1,), (1,)), ((), ())),
        preferred_element_type=jnp.float32)
    d = (x2 + e2[None, :]) - 2.0 * mm
    idx = jnp.argmin(d, axis=1).astype(jnp.int32)
    code_iota = jax.lax.broadcasted_iota(jnp.int32, d.shape, 1)
    onehot = (code_iota == idx[:, None]).astype(jnp.float32)
    q = jax.lax.dot_general(
        onehot, emb, (((1,), (0,)), ((), ())),
        preferred_element_type=jnp.float32)
    idx_ref[0, 0, :] = idx
    q_ref[...] = x + (q - x)
    diff = x - q
    part = jnp.sum(diff * diff)

    @pl.when(pl.program_id(0) == 0)
    def _():
        loss_ref[0, 0] = 0.0

    loss_ref[0, 0] += part


def kernel(x, emb):
    b, s, d = x.shape
    n = b * s
    tile = 1024
    grid = n // tile
    x_flat = x.reshape(n, d)
    q_flat, idx2d, loss_sum = pl.pallas_call(
        _vq_tile_kernel,
        grid=(grid,),
        in_specs=[
            pl.BlockSpec((tile, d), lambda i: (i, 0)),
            pl.BlockSpec((_CODES, d), lambda i: (0, 0)),
        ],
        out_specs=[
            pl.BlockSpec((tile, d), lambda i: (i, 0)),
            pl.BlockSpec((1, 1, tile), lambda i: (i, 0, 0)),
            pl.BlockSpec(memory_space=pltpu.SMEM),
        ],
        out_shape=[
            jax.ShapeDtypeStruct((n, d), jnp.float32),
            jax.ShapeDtypeStruct((grid, 1, tile), jnp.int32),
            jax.ShapeDtypeStruct((1, 1), jnp.float32),
        ],
    )(x_flat, emb)
    quantized = q_flat.reshape(b, s, d)
    indices = idx2d.reshape(b, s)
    m = loss_sum[0, 0] / jnp.float32(n * d)
    loss = m + _COMMIT_W * m
    return (quantized, indices, loss)
